# baseline (speedup 1.0000x reference)
import jax
import jax.numpy as jnp
from jax.experimental import pallas as pl
from jax.experimental.pallas import tpu as pltpu

TOTAL = 16384
D = 2048
SEG = 2048
NSEG = TOTAL // SEG
BM = 1024
M_TILES = SEG // BM

_NT = (((1,), (1,)), ((), ()))


def _fused_body(x_ref, w_ref, b_ref, o_ref, c_ref):
    m = pl.program_id(1)

    @pl.when(m == 0)
    def _():
        xm = jnp.mean(x_ref[...], axis=0, keepdims=True).astype(jnp.bfloat16)
        c_ref[...] = b_ref[...] - jax.lax.dot_general(
            xm, w_ref[...], dimension_numbers=_NT,
            preferred_element_type=jnp.float32)

    xc = x_ref[pl.ds(m * BM, BM), :].astype(jnp.bfloat16)
    o_ref[...] = jax.lax.dot_general(
        xc, w_ref[...], dimension_numbers=_NT,
        preferred_element_type=jnp.float32,
    ) + c_ref[...]


def kernel(x, W, b, l):
    b_eff = (b + (jnp.asarray(l) - SEG).astype(jnp.float32)).reshape(1, D)
    W_bf = W.astype(jnp.bfloat16)

    out = pl.pallas_call(
        _fused_body,
        grid=(NSEG, M_TILES),
        in_specs=[
            pl.BlockSpec((SEG, D), lambda s, m: (s, 0)),
            pl.BlockSpec((D, D), lambda s, m: (0, 0)),
            pl.BlockSpec((1, D), lambda s, m: (0, 0)),
        ],
        out_specs=pl.BlockSpec((BM, D), lambda s, m: (s * M_TILES + m, 0)),
        out_shape=jax.ShapeDtypeStruct((TOTAL, D), jnp.float32),
        scratch_shapes=[pltpu.VMEM((1, D), jnp.float32)],
        compiler_params=pltpu.CompilerParams(
            vmem_limit_bytes=64 * 1024 * 1024,
        ),
    )(x, W_bf, b_eff)
    return out


# speedup vs baseline: 1.0307x; 1.0307x over previous
import jax
import jax.numpy as jnp
from jax.experimental import pallas as pl
from jax.experimental.pallas import tpu as pltpu

TOTAL = 16384
D = 2048
SEG = 2048
NSEG = TOTAL // SEG
BM = 1024
N_BLOCKS = TOTAL // BM

_NT = (((1,), (1,)), ((), ()))


def _body(x_ref, w_ref, b_ref, o_ref, bufa_ref, bufb_ref, sum_ref):
    q = pl.program_id(0)
    odd = q % 2 == 1

    colsum = jnp.sum(x_ref[...], axis=0, keepdims=True)

    @pl.when(jnp.logical_and(~odd, q < N_BLOCKS))
    def _():
        sum_ref[...] = colsum
        bufa_ref[...] = x_ref[...]

    @pl.when(odd)
    def _():
        xm = (sum_ref[...] + colsum) * (1.0 / SEG)
        xca = (bufa_ref[...] - xm).astype(jnp.bfloat16)
        o_ref[...] = jax.lax.dot_general(
            xca, w_ref[...], dimension_numbers=_NT,
            preferred_element_type=jnp.float32) + b_ref[...]
        bufb_ref[...] = (x_ref[...] - xm).astype(jnp.bfloat16)

    @pl.when(jnp.logical_and(~odd, q > 0))
    def _():
        o_ref[...] = jax.lax.dot_general(
            bufb_ref[...], w_ref[...], dimension_numbers=_NT,
            preferred_element_type=jnp.float32) + b_ref[...]


def kernel(x, W, b, l):
    b_eff = (b + (jnp.asarray(l) - SEG).astype(jnp.float32)).reshape(1, D)
    W_bf = W.astype(jnp.bfloat16)

    out = pl.pallas_call(
        _body,
        grid=(N_BLOCKS + 1,),
        in_specs=[
            pl.BlockSpec((BM, D), lambda q: (jnp.minimum(q, N_BLOCKS - 1), 0)),
            pl.BlockSpec((D, D), lambda q: (0, 0)),
            pl.BlockSpec((1, D), lambda q: (0, 0)),
        ],
        out_specs=pl.BlockSpec(
            (BM, D), lambda q: (jnp.maximum(q - 1, 0), 0)),
        out_shape=jax.ShapeDtypeStruct((TOTAL, D), jnp.float32),
        scratch_shapes=[
            pltpu.VMEM((BM, D), jnp.float32),
            pltpu.VMEM((BM, D), jnp.bfloat16),
            pltpu.VMEM((1, D), jnp.float32),
        ],
        compiler_params=pltpu.CompilerParams(
            vmem_limit_bytes=64 * 1024 * 1024,
        ),
    )(x, W_bf, b_eff)
    return out


# speedup vs baseline: 1.0938x; 1.0612x over previous
import jax
import jax.numpy as jnp
from jax.experimental import pallas as pl
from jax.experimental.pallas import tpu as pltpu

TOTAL = 16384
D = 2048
SEG = 2048
NSEG = TOTAL // SEG
BM = 1024
M_TILES = SEG // BM


def _fused_body(x_ref, w_ref, b_ref, o_ref, xm_ref):
    m = pl.program_id(1)

    @pl.when(m == 0)
    def _():
        xm_ref[...] = jnp.mean(x_ref[...], axis=0, keepdims=True)

    xc = (x_ref[pl.ds(m * BM, BM), :] - xm_ref[...]).astype(jnp.bfloat16)
    o_ref[...] = jax.lax.dot_general(
        xc, w_ref[...],
        dimension_numbers=(((1,), (1,)), ((), ())),
        preferred_element_type=jnp.float32,
    ) + b_ref[...]


def kernel(x, W, b, l):
    b_eff = (b + (jnp.asarray(l) - SEG).astype(jnp.float32)).reshape(1, D)
    W_bf = W.astype(jnp.bfloat16)

    out = pl.pallas_call(
        _fused_body,
        grid=(NSEG, M_TILES),
        in_specs=[
            pl.BlockSpec((SEG, D), lambda s, m: (s, 0)),
            pl.BlockSpec((D, D), lambda s, m: (0, 0)),
            pl.BlockSpec((1, D), lambda s, m: (0, 0)),
        ],
        out_specs=pl.BlockSpec((BM, D), lambda s, m: (s * M_TILES + m, 0)),
        out_shape=jax.ShapeDtypeStruct((TOTAL, D), jnp.float32),
        scratch_shapes=[pltpu.VMEM((1, D), jnp.float32)],
        compiler_params=pltpu.CompilerParams(
            vmem_limit_bytes=64 * 1024 * 1024,
        ),
    )(x, W_bf, b_eff)
    return out
